# Initial kernel scaffold; baseline (speedup 1.0000x reference)
import functools

import jax
import jax.numpy as jnp
from jax import lax
from jax.experimental import pallas as pl
from jax.experimental.pallas import tpu as pltpu
from jax.experimental.pallas import tpu_sc as plsc

N = 10000
NP = 10240
E = 320000
NC = 2
NS = 16
NW = NC * NS
C = 128
NCHUNK = 79
EPW = C * NCHUNK
EPAD = EPW * NW
STRIPE = NP // NS
DIN = 128
DHID = 128
DOUT = 47
DOP = 48
NT = 1000
NTP = 1024
BN = 512

_F32 = jnp.float32


def _vmesh():
    return plsc.VectorSubcoreMesh(core_axis_name="c", subcore_axis_name="s")


def _mm(x, w):
    m, k = x.shape
    _, n = w.shape

    def body(x_ref, w_ref, o_ref):
        o_ref[...] = jnp.dot(x_ref[...], w_ref[...],
                             preferred_element_type=_F32,
                             precision=lax.Precision.HIGHEST)

    return pl.pallas_call(
        body,
        grid=(m // BN,),
        in_specs=[pl.BlockSpec((BN, k), lambda i: (i, 0)),
                  pl.BlockSpec((k, n), lambda i: (0, 0))],
        out_specs=pl.BlockSpec((BN, n), lambda i: (i, 0)),
        out_shape=jax.ShapeDtypeStruct((m, n), _F32),
    )(x, w)


def _agg_deg(table, srcp, dstp):
    d = table.shape[1]
    zrow = jnp.zeros((STRIPE, d), _F32)
    zdeg = jnp.zeros((STRIPE, 16), _F32)
    onesc = jnp.ones((C, 16), _F32)

    @functools.partial(
        pl.kernel,
        out_type=[jax.ShapeDtypeStruct((NC, NP, d), _F32),
                  jax.ShapeDtypeStruct((NC, NP, 16), _F32)],
        mesh=_vmesh(),
        scratch_types=[pltpu.VMEM((C,), jnp.int32),
                       pltpu.VMEM((C,), jnp.int32),
                       pltpu.VMEM((C, d), _F32),
                       pltpu.VMEM((C, 16), _F32),
                       pltpu.VMEM_SHARED((NP, d), _F32),
                       pltpu.VMEM_SHARED((NP, 16), _F32),
                       pltpu.SemaphoreType.DMA],
    )
    def k(tab_hbm, src_hbm, dst_hbm, zrow_hbm, zdeg_hbm, ones_hbm,
          accf_o, accd_o, sidx, didx, buf, onesb, accsh, degsh, sem):
        c = lax.axis_index("c")
        s = lax.axis_index("s")
        w = c * NS + s
        row = pl.ds(s * STRIPE, STRIPE)
        pltpu.sync_copy(zrow_hbm, accsh.at[row])
        pltpu.sync_copy(zdeg_hbm, degsh.at[row])
        pltpu.sync_copy(ones_hbm, onesb)
        plsc.subcore_barrier()

        base = w * EPW

        @pl.loop(0, NCHUNK)
        def _(j):
            off = base + j * C
            pltpu.sync_copy(src_hbm.at[pl.ds(off, C)], sidx)
            pltpu.sync_copy(dst_hbm.at[pl.ds(off, C)], didx)
            pltpu.async_copy(tab_hbm.at[sidx], buf, sem).wait()
            pltpu.sync_copy(buf, accsh.at[didx], add=True)
            pltpu.sync_copy(onesb, degsh.at[didx], add=True)

        plsc.subcore_barrier()
        pltpu.sync_copy(accsh.at[row], accf_o.at[c].at[row])
        pltpu.sync_copy(degsh.at[row], accd_o.at[c].at[row])

    return k(table, srcp, dstp, zrow, zdeg, onesc)


def _agg(table, srcp, dstp):
    d = table.shape[1]
    zrow = jnp.zeros((STRIPE, d), _F32)

    @functools.partial(
        pl.kernel,
        out_type=jax.ShapeDtypeStruct((NC, NP, d), _F32),
        mesh=_vmesh(),
        scratch_types=[pltpu.VMEM((C,), jnp.int32),
                       pltpu.VMEM((C,), jnp.int32),
                       pltpu.VMEM((C, d), _F32),
                       pltpu.VMEM_SHARED((NP, d), _F32),
                       pltpu.SemaphoreType.DMA],
    )
    def k(tab_hbm, src_hbm, dst_hbm, zrow_hbm, accf_o,
          sidx, didx, buf, accsh, sem):
        c = lax.axis_index("c")
        s = lax.axis_index("s")
        w = c * NS + s
        row = pl.ds(s * STRIPE, STRIPE)
        pltpu.sync_copy(zrow_hbm, accsh.at[row])
        plsc.subcore_barrier()

        base = w * EPW

        @pl.loop(0, NCHUNK)
        def _(j):
            off = base + j * C
            pltpu.sync_copy(src_hbm.at[pl.ds(off, C)], sidx)
            pltpu.sync_copy(dst_hbm.at[pl.ds(off, C)], didx)
            pltpu.async_copy(tab_hbm.at[sidx], buf, sem).wait()
            pltpu.sync_copy(buf, accsh.at[didx], add=True)

        plsc.subcore_barrier()
        pltpu.sync_copy(accsh.at[row], accf_o.at[c].at[row])

    return k(table, srcp, dstp, zrow)


def _layer1(a0, a1, d0, d1, z1, b1r, w2p):
    def body(a0_r, a1_r, d0_r, d1_r, z_r, b_r, w_r, y_o, r_o):
        deg = d0_r[...][:, :1] + d1_r[...][:, :1]
        r = 1.0 / (deg + 1.0)
        h1 = jnp.maximum((a0_r[...] + a1_r[...] + z_r[...]) * r + b_r[...],
                         0.0)
        y_o[...] = jnp.dot(h1, w_r[...], preferred_element_type=_F32,
                           precision=lax.Precision.HIGHEST)
        r_o[...] = jnp.broadcast_to(r, (BN, DOP))

    return pl.pallas_call(
        body,
        grid=(NP // BN,),
        in_specs=[pl.BlockSpec((BN, DHID), lambda i: (i, 0)),
                  pl.BlockSpec((BN, DHID), lambda i: (i, 0)),
                  pl.BlockSpec((BN, 16), lambda i: (i, 0)),
                  pl.BlockSpec((BN, 16), lambda i: (i, 0)),
                  pl.BlockSpec((BN, DHID), lambda i: (i, 0)),
                  pl.BlockSpec((1, DHID), lambda i: (0, 0)),
                  pl.BlockSpec((DHID, DOP), lambda i: (0, 0))],
        out_specs=[pl.BlockSpec((BN, DOP), lambda i: (i, 0)),
                   pl.BlockSpec((BN, DOP), lambda i: (i, 0))],
        out_shape=[jax.ShapeDtypeStruct((NP, DOP), _F32),
                   jax.ShapeDtypeStruct((NP, DOP), _F32)],
    )(a0, a1, d0, d1, z1, b1r, w2p)


def _layer2_softmax(a0, a1, y2, rr, b2p, labf):
    def body(a0_r, a1_r, y_r, r_r, b_r, l_r, o_r):
        h2 = (a0_r[...] + a1_r[...] + y_r[...]) * r_r[...] + b_r[...]
        col = lax.broadcasted_iota(jnp.int32, (BN, DOP), 1)
        x = jnp.where(col < DOUT, h2, -1e30)
        m = jnp.max(x, axis=1, keepdims=True)
        ls = (x - m) - jnp.log(jnp.sum(jnp.exp(x - m), axis=1,
                                       keepdims=True))
        o_r[...] = jnp.concatenate([ls, l_r[...]], axis=1)

    return pl.pallas_call(
        body,
        grid=(NP // BN,),
        in_specs=[pl.BlockSpec((BN, DOP), lambda i: (i, 0)),
                  pl.BlockSpec((BN, DOP), lambda i: (i, 0)),
                  pl.BlockSpec((BN, DOP), lambda i: (i, 0)),
                  pl.BlockSpec((BN, DOP), lambda i: (i, 0)),
                  pl.BlockSpec((1, DOP), lambda i: (0, 0)),
                  pl.BlockSpec((BN, 16), lambda i: (i, 0))],
        out_specs=pl.BlockSpec((BN, DOP + 16), lambda i: (i, 0)),
        out_shape=jax.ShapeDtypeStruct((NP, DOP + 16), _F32),
    )(a0, a1, y2, rr, b2p, labf)


def _select(t3, tnp):
    @functools.partial(
        pl.kernel,
        out_type=jax.ShapeDtypeStruct((NTP, DOP + 16), _F32),
        mesh=_vmesh(),
        scratch_types=[pltpu.VMEM((NTP // NW,), jnp.int32),
                       pltpu.VMEM((NTP // NW, DOP + 16), _F32),
                       pltpu.SemaphoreType.DMA],
    )
    def k(t3_hbm, tn_hbm, o_hbm, idx, buf, sem):
        c = lax.axis_index("c")
        s = lax.axis_index("s")
        w = c * NS + s
        bs = NTP // NW
        sl = pl.ds(w * bs, bs)
        pltpu.sync_copy(tn_hbm.at[sl], idx)
        pltpu.async_copy(t3_hbm.at[idx], buf, sem).wait()
        pltpu.sync_copy(buf, o_hbm.at[sl])

    return k(t3, tnp)


def _loss(sel):
    def body(s_ref, o_ref):
        s = s_ref[...]
        ls = s[:, :DOP]
        lbl = s[:, DOP:DOP + 1].astype(jnp.int32)
        col = lax.broadcasted_iota(jnp.int32, (NTP, DOP), 1)
        picked = jnp.sum(jnp.where(col == lbl, ls, 0.0), axis=1,
                         keepdims=True)
        rowv = lax.broadcasted_iota(jnp.int32, (NTP, 1), 0) < NT
        tot = jnp.sum(jnp.where(rowv, picked, 0.0), axis=0, keepdims=True)
        o_ref[...] = -tot / NT

    return pl.pallas_call(
        body,
        out_shape=jax.ShapeDtypeStruct((1, 1), _F32),
    )(sel)


def kernel(feats, edge_index, label, train_nodes, W1, b1, W2, b2):
    src = edge_index[0]
    dst = edge_index[1]
    pad = jnp.full((EPAD - E,), NP - 1, jnp.int32)
    srcp = jnp.concatenate([src, pad])
    dstp = jnp.concatenate([dst, pad])

    featsp = jnp.zeros((NP, DIN), _F32).at[:N].set(feats)
    w2p = jnp.zeros((DIN, DOP), _F32).at[:, :DOUT].set(W2)
    b1r = b1[None, :]
    b2p = jnp.zeros((1, DOP), _F32).at[0, :DOUT].set(b2)
    labf = jnp.broadcast_to(
        jnp.zeros((NP,), _F32).at[:N].set(label.astype(_F32))[:, None],
        (NP, 16))
    tnp = jnp.concatenate(
        [train_nodes, jnp.zeros((NTP - NT,), jnp.int32)])

    z1 = _mm(featsp, W1)
    accf, accd = _agg_deg(z1, srcp, dstp)
    y2, rr = _layer1(accf[0], accf[1], accd[0], accd[1], z1, b1r, w2p)
    acc2 = _agg(y2, srcp, dstp)
    t3 = _layer2_softmax(acc2[0], acc2[1], y2, rr, b2p, labf)
    sel = _select(t3, tnp)
    loss = _loss(sel)[0, 0]

    h_out = t3[:N, :DOUT]
    return (h_out, loss)


# speedup vs baseline: 4.5789x; 4.5789x over previous
import functools

import jax
import jax.numpy as jnp
from jax import lax
from jax.experimental import pallas as pl
from jax.experimental.pallas import tpu as pltpu
from jax.experimental.pallas import tpu_sc as plsc

N = 10000
NP = 10240
E = 320000
NC = 2
NS = 16
NW = NC * NS
C = 128
NCHUNK = 79
EPW = C * NCHUNK
EPAD = EPW * NW
STRIPE = NP // NS
DIN = 128
DHID = 128
DOUT = 47
DOP = 48
NT = 1000
NTP = 1024
BN = 512

_F32 = jnp.float32


def _vmesh():
    return plsc.VectorSubcoreMesh(core_axis_name="c", subcore_axis_name="s")


_SC_PARAMS = pltpu.CompilerParams(use_tc_tiling_on_sc=False)


def _mm(x, w):
    m, k = x.shape
    _, n = w.shape

    def body(x_ref, w_ref, o_ref):
        o_ref[...] = jnp.dot(x_ref[...], w_ref[...],
                             preferred_element_type=_F32,
                             precision=lax.Precision.HIGHEST)

    return pl.pallas_call(
        body,
        grid=(m // BN,),
        in_specs=[pl.BlockSpec((BN, k), lambda i: (i, 0)),
                  pl.BlockSpec((k, n), lambda i: (0, 0))],
        out_specs=pl.BlockSpec((BN, n), lambda i: (i, 0)),
        out_shape=jax.ShapeDtypeStruct((m, n), _F32),
    )(x, w)


def _agg_deg(table, srcp, dstp):
    d = table.shape[1]
    zrow = jnp.zeros((STRIPE, d), _F32)
    zdeg = jnp.zeros((STRIPE, 16), _F32)
    onesc = jnp.ones((C, 16), _F32)

    @functools.partial(
        pl.kernel,
        out_type=[jax.ShapeDtypeStruct((NC, NP, d), _F32),
                  jax.ShapeDtypeStruct((NC, NP, 16), _F32)],
        mesh=_vmesh(),
        compiler_params=_SC_PARAMS,
        scratch_types=[pltpu.VMEM((C,), jnp.int32),
                       pltpu.VMEM((C,), jnp.int32),
                       pltpu.VMEM((C, d), _F32),
                       pltpu.VMEM((C, 16), _F32),
                       pltpu.VMEM_SHARED((NP, d), _F32),
                       pltpu.VMEM_SHARED((NP, 16), _F32),
                       pltpu.SemaphoreType.DMA],
    )
    def k(tab_hbm, src_hbm, dst_hbm, zrow_hbm, zdeg_hbm, ones_hbm,
          accf_o, accd_o, sidx, didx, buf, onesb, accsh, degsh, sem):
        c = lax.axis_index("c")
        s = lax.axis_index("s")
        w = c * NS + s
        row = pl.ds(s * STRIPE, STRIPE)
        pltpu.sync_copy(zrow_hbm, accsh.at[row])
        pltpu.sync_copy(zdeg_hbm, degsh.at[row])
        pltpu.sync_copy(ones_hbm, onesb)
        plsc.subcore_barrier()

        base = w * EPW

        @pl.loop(0, NCHUNK)
        def _(j):
            off = base + j * C
            pltpu.sync_copy(src_hbm.at[pl.ds(off, C)], sidx)
            pltpu.sync_copy(dst_hbm.at[pl.ds(off, C)], didx)
            pltpu.async_copy(tab_hbm.at[sidx], buf, sem).wait()
            pltpu.sync_copy(buf, accsh.at[didx], add=True)
            pltpu.sync_copy(onesb, degsh.at[didx], add=True)

        plsc.subcore_barrier()
        pltpu.sync_copy(accsh.at[row], accf_o.at[c].at[row])
        pltpu.sync_copy(degsh.at[row], accd_o.at[c].at[row])

    return k(table, srcp, dstp, zrow, zdeg, onesc)


def _agg(table, srcp, dstp):
    d = table.shape[1]
    zrow = jnp.zeros((STRIPE, d), _F32)

    @functools.partial(
        pl.kernel,
        out_type=jax.ShapeDtypeStruct((NC, NP, d), _F32),
        mesh=_vmesh(),
        compiler_params=_SC_PARAMS,
        scratch_types=[pltpu.VMEM((C,), jnp.int32),
                       pltpu.VMEM((C,), jnp.int32),
                       pltpu.VMEM((C, d), _F32),
                       pltpu.VMEM_SHARED((NP, d), _F32),
                       pltpu.SemaphoreType.DMA],
    )
    def k(tab_hbm, src_hbm, dst_hbm, zrow_hbm, accf_o,
          sidx, didx, buf, accsh, sem):
        c = lax.axis_index("c")
        s = lax.axis_index("s")
        w = c * NS + s
        row = pl.ds(s * STRIPE, STRIPE)
        pltpu.sync_copy(zrow_hbm, accsh.at[row])
        plsc.subcore_barrier()

        base = w * EPW

        @pl.loop(0, NCHUNK)
        def _(j):
            off = base + j * C
            pltpu.sync_copy(src_hbm.at[pl.ds(off, C)], sidx)
            pltpu.sync_copy(dst_hbm.at[pl.ds(off, C)], didx)
            pltpu.async_copy(tab_hbm.at[sidx], buf, sem).wait()
            pltpu.sync_copy(buf, accsh.at[didx], add=True)

        plsc.subcore_barrier()
        pltpu.sync_copy(accsh.at[row], accf_o.at[c].at[row])

    return k(table, srcp, dstp, zrow)


def _layer1(a0, a1, d0, d1, z1, b1r, w2p):
    def body(a0_r, a1_r, d0_r, d1_r, z_r, b_r, w_r, y_o, r_o):
        deg = d0_r[...][:, :1] + d1_r[...][:, :1]
        r = 1.0 / (deg + 1.0)
        h1 = jnp.maximum((a0_r[...] + a1_r[...] + z_r[...]) * r + b_r[...],
                         0.0)
        y_o[...] = jnp.dot(h1, w_r[...], preferred_element_type=_F32,
                           precision=lax.Precision.HIGHEST)
        r_o[...] = jnp.broadcast_to(r, (BN, DOP))

    return pl.pallas_call(
        body,
        grid=(NP // BN,),
        in_specs=[pl.BlockSpec((BN, DHID), lambda i: (i, 0)),
                  pl.BlockSpec((BN, DHID), lambda i: (i, 0)),
                  pl.BlockSpec((BN, 16), lambda i: (i, 0)),
                  pl.BlockSpec((BN, 16), lambda i: (i, 0)),
                  pl.BlockSpec((BN, DHID), lambda i: (i, 0)),
                  pl.BlockSpec((1, DHID), lambda i: (0, 0)),
                  pl.BlockSpec((DHID, DOP), lambda i: (0, 0))],
        out_specs=[pl.BlockSpec((BN, DOP), lambda i: (i, 0)),
                   pl.BlockSpec((BN, DOP), lambda i: (i, 0))],
        out_shape=[jax.ShapeDtypeStruct((NP, DOP), _F32),
                   jax.ShapeDtypeStruct((NP, DOP), _F32)],
    )(a0, a1, d0, d1, z1, b1r, w2p)


def _layer2_softmax(a0, a1, y2, rr, b2p, labf):
    def body(a0_r, a1_r, y_r, r_r, b_r, l_r, o_r):
        h2 = (a0_r[...] + a1_r[...] + y_r[...]) * r_r[...] + b_r[...]
        col = lax.broadcasted_iota(jnp.int32, (BN, DOP), 1)
        x = jnp.where(col < DOUT, h2, -1e30)
        m = jnp.max(x, axis=1, keepdims=True)
        ls = (x - m) - jnp.log(jnp.sum(jnp.exp(x - m), axis=1,
                                       keepdims=True))
        o_r[...] = jnp.concatenate([ls, l_r[...]], axis=1)

    return pl.pallas_call(
        body,
        grid=(NP // BN,),
        in_specs=[pl.BlockSpec((BN, DOP), lambda i: (i, 0)),
                  pl.BlockSpec((BN, DOP), lambda i: (i, 0)),
                  pl.BlockSpec((BN, DOP), lambda i: (i, 0)),
                  pl.BlockSpec((BN, DOP), lambda i: (i, 0)),
                  pl.BlockSpec((1, DOP), lambda i: (0, 0)),
                  pl.BlockSpec((BN, 16), lambda i: (i, 0))],
        out_specs=pl.BlockSpec((BN, DOP + 16), lambda i: (i, 0)),
        out_shape=jax.ShapeDtypeStruct((NP, DOP + 16), _F32),
    )(a0, a1, y2, rr, b2p, labf)


def _select(t3, tnp):
    @functools.partial(
        pl.kernel,
        out_type=jax.ShapeDtypeStruct((NTP, DOP + 16), _F32),
        mesh=_vmesh(),
        compiler_params=_SC_PARAMS,
        scratch_types=[pltpu.VMEM((NTP // NW,), jnp.int32),
                       pltpu.VMEM((NTP // NW, DOP + 16), _F32),
                       pltpu.SemaphoreType.DMA],
    )
    def k(t3_hbm, tn_hbm, o_hbm, idx, buf, sem):
        c = lax.axis_index("c")
        s = lax.axis_index("s")
        w = c * NS + s
        bs = NTP // NW
        sl = pl.ds(w * bs, bs)
        pltpu.sync_copy(tn_hbm.at[sl], idx)
        pltpu.async_copy(t3_hbm.at[idx], buf, sem).wait()
        pltpu.sync_copy(buf, o_hbm.at[sl])

    return k(t3, tnp)


def _loss(sel):
    def body(s_ref, o_ref):
        s = s_ref[...]
        ls = s[:, :DOP]
        lbl = s[:, DOP:DOP + 1].astype(jnp.int32)
        col = lax.broadcasted_iota(jnp.int32, (NTP, DOP), 1)
        picked = jnp.sum(jnp.where(col == lbl, ls, 0.0), axis=1,
                         keepdims=True)
        rowv = lax.broadcasted_iota(jnp.int32, (NTP, 1), 0) < NT
        tot = jnp.sum(jnp.where(rowv, picked, 0.0), axis=0, keepdims=True)
        o_ref[...] = -tot / NT

    return pl.pallas_call(
        body,
        out_shape=jax.ShapeDtypeStruct((1, 1), _F32),
    )(sel)


def kernel(feats, edge_index, label, train_nodes, W1, b1, W2, b2):
    src = edge_index[0]
    dst = edge_index[1]
    pad = jnp.full((EPAD - E,), NP - 1, jnp.int32)
    srcp = jnp.concatenate([src, pad])
    dstp = jnp.concatenate([dst, pad])

    featsp = jnp.zeros((NP, DIN), _F32).at[:N].set(feats)
    w2p = jnp.zeros((DIN, DOP), _F32).at[:, :DOUT].set(W2)
    b1r = b1[None, :]
    b2p = jnp.zeros((1, DOP), _F32).at[0, :DOUT].set(b2)
    labf = jnp.broadcast_to(
        jnp.zeros((NP,), _F32).at[:N].set(label.astype(_F32))[:, None],
        (NP, 16))
    tnp = jnp.concatenate(
        [train_nodes, jnp.zeros((NTP - NT,), jnp.int32)])

    z1 = _mm(featsp, W1)
    accf, accd = _agg_deg(z1, srcp, dstp)
    y2, rr = _layer1(accf[0], accf[1], accd[0], accd[1], z1, b1r, w2p)
    acc2 = _agg(y2, srcp, dstp)
    t3 = _layer2_softmax(acc2[0], acc2[1], y2, rr, b2p, labf)
    sel = _select(t3, tnp)
    loss = _loss(sel)[0, 0]

    h_out = t3[:N, :DOUT]
    return (h_out, loss)


# speedup vs baseline: 13.1824x; 2.8789x over previous
import functools

import jax
import jax.numpy as jnp
from jax import lax
from jax.experimental import pallas as pl
from jax.experimental.pallas import tpu as pltpu
from jax.experimental.pallas import tpu_sc as plsc

N = 10000
E = 320000
NC = 2
NS = 16
NW = NC * NS
C = 128
NCH = E // C
NCHW = 78
NTAIL = NCH - NCHW * NW
STRIPE = N // NS
DIN = 128
DHID = 128
DOUT = 47
DOP = 48
NT = 1000
NTP = 1024
BM = 2000

_F32 = jnp.float32


def _vmesh():
    return plsc.VectorSubcoreMesh(core_axis_name="c", subcore_axis_name="s")


_SC_PARAMS = pltpu.CompilerParams(use_tc_tiling_on_sc=False)


def _agg_call(table, edges, with_deg):
    d = table.shape[1]
    scratch = ([pltpu.VMEM((2, C), jnp.int32)] * 6
               + [pltpu.VMEM((C, d), _F32)] * 2
               + [pltpu.VMEM_SHARED((N, d), _F32)]
               + [pltpu.SemaphoreType.DMA] * 10)
    out_type = [jax.ShapeDtypeStruct((NC, N, d), _F32)]
    ins = [table, edges, jnp.zeros((STRIPE, d), _F32)]
    if with_deg:
        scratch += [pltpu.VMEM((C, 16), _F32),
                    pltpu.VMEM_SHARED((N, 16), _F32),
                    pltpu.SemaphoreType.DMA,
                    pltpu.SemaphoreType.DMA,
                    pltpu.SemaphoreType.DMA]
        out_type.append(jax.ShapeDtypeStruct((NC, N, 16), _F32))
        ins += [jnp.zeros((STRIPE, 16), _F32), jnp.ones((C, 16), _F32)]

    def body(*refs):
        it = iter(refs)
        tab = next(it)
        edge_h = next(it)
        zrow_h = next(it)
        if with_deg:
            zdeg_h = next(it)
            ones_h = next(it)
        accf_o = next(it)
        accd_o = next(it) if with_deg else None
        islot = [next(it) for _ in range(6)]
        bufs = [next(it) for _ in range(2)]
        accsh = next(it)
        isem = [next(it) for _ in range(6)]
        gsem = [next(it) for _ in range(2)]
        ssem = [next(it) for _ in range(2)]
        if with_deg:
            onesb = next(it)
            degsh = next(it)
            dsem = [next(it) for _ in range(3)]

        c = lax.axis_index("c")
        s = lax.axis_index("s")
        w = c * NS + s
        row = pl.ds(s * STRIPE, STRIPE)
        pltpu.sync_copy(zrow_h, accsh.at[row])
        if with_deg:
            pltpu.sync_copy(zdeg_h, degsh.at[row])
            pltpu.sync_copy(ones_h, onesb)
        plsc.subcore_barrier()

        def i_cp(t, sl):
            off = (t * NW + w) * C
            return pltpu.make_async_copy(edge_h.at[:, pl.ds(off, C)],
                                         islot[sl], isem[sl])

        def g_cp(sl, b):
            return pltpu.make_async_copy(tab.at[islot[sl].at[0]], bufs[b],
                                         gsem[b])

        def s_cp(sl, b):
            return pltpu.make_async_copy(bufs[b], accsh.at[islot[sl].at[1]],
                                         ssem[b])

        def d_cp(sl, b3):
            return pltpu.make_async_copy(onesb, degsh.at[islot[sl].at[1]],
                                         dsem[b3])

        i_cp(0, 0).start()
        i_cp(1, 1).start()
        i_cp(2, 2).start()
        i_cp(0, 0).wait()
        g_cp(0, 0).start()
        i_cp(1, 1).wait()
        g_cp(1, 1).start()

        @pl.loop(0, NCHW // 6)
        def _(g):
            for b6 in range(6):
                bb = b6 % 2
                b3 = b6 % 3
                t = g * 6 + b6
                g_cp(b6, bb).wait()
                s_cp(b6, bb).start(add=True)
                if with_deg:
                    @pl.when(t >= 3)
                    def _():
                        d_cp(b6, b3).wait()

                    d_cp(b6, b3).start(add=True)
                s_cp(b6, bb).wait()

                @pl.when(t + 3 < NCHW)
                def _():
                    i_cp(t + 3, (b6 + 3) % 6).start()

                @pl.when(t + 2 < NCHW)
                def _():
                    i_cp(t + 2, (b6 + 2) % 6).wait()
                    g_cp((b6 + 2) % 6, bb).start()

        if with_deg:
            for b3 in range(3):
                d_cp(0, b3).wait()

        @pl.when(w < NTAIL)
        def _():
            i_cp(NCHW, 0).start()
            i_cp(NCHW, 0).wait()
            g_cp(0, 0).start()
            g_cp(0, 0).wait()
            s_cp(0, 0).start(add=True)
            s_cp(0, 0).wait()
            if with_deg:
                d_cp(0, 0).start(add=True)
                d_cp(0, 0).wait()

        plsc.subcore_barrier()
        pltpu.sync_copy(accsh.at[row], accf_o.at[c].at[row])
        if with_deg:
            pltpu.sync_copy(degsh.at[row], accd_o.at[c].at[row])

    k = pl.kernel(body, out_type=out_type, mesh=_vmesh(),
                  compiler_params=_SC_PARAMS, scratch_types=scratch)
    res = k(*ins)
    return (res[0], res[1]) if with_deg else res[0]


def _agg_deg(table, edges):
    return _agg_call(table, edges, True)


def _agg(table, edges):
    return _agg_call(table, edges, False)


def _layer1(accf, accd, feats, b1r, w1, w2p):
    def body(a0_r, a1_r, d0_r, d1_r, f_r, b_r, w1_r, w2_r, y_o, r_o):
        deg = d0_r[0][:, :1] + d1_r[0][:, :1]
        r = 1.0 / (deg + 1.0)
        t = (a0_r[0] + a1_r[0] + f_r[...]) * r
        h1 = jnp.maximum(jnp.dot(t, w1_r[...], preferred_element_type=_F32)
                         + b_r[...], 0.0)
        y_o[...] = jnp.dot(h1, w2_r[...], preferred_element_type=_F32)
        r_o[...] = jnp.broadcast_to(r, (BM, DOP))

    return pl.pallas_call(
        body,
        grid=(N // BM,),
        in_specs=[pl.BlockSpec((1, BM, DHID), lambda i: (0, i, 0)),
                  pl.BlockSpec((1, BM, DHID), lambda i: (1, i, 0)),
                  pl.BlockSpec((1, BM, 16), lambda i: (0, i, 0)),
                  pl.BlockSpec((1, BM, 16), lambda i: (1, i, 0)),
                  pl.BlockSpec((BM, DIN), lambda i: (i, 0)),
                  pl.BlockSpec((1, DHID), lambda i: (0, 0)),
                  pl.BlockSpec((DIN, DHID), lambda i: (0, 0)),
                  pl.BlockSpec((DHID, DOP), lambda i: (0, 0))],
        out_specs=[pl.BlockSpec((BM, DOP), lambda i: (i, 0)),
                   pl.BlockSpec((BM, DOP), lambda i: (i, 0))],
        out_shape=[jax.ShapeDtypeStruct((N, DOP), _F32),
                   jax.ShapeDtypeStruct((N, DOP), _F32)],
    )(accf, accf, accd, accd, feats, b1r, w1, w2p)


def _layer2_softmax(acc2, y2, rr, b2p, labf):
    def body(a0_r, a1_r, y_r, r_r, b_r, l_r, o_r, h_r):
        h2 = (a0_r[0] + a1_r[0] + y_r[...]) * r_r[...] + b_r[...]
        col = lax.broadcasted_iota(jnp.int32, (BM, DOP), 1)
        x = jnp.where(col < DOUT, h2, -1e30)
        m = jnp.max(x, axis=1, keepdims=True)
        ls = (x - m) - jnp.log(jnp.sum(jnp.exp(x - m), axis=1,
                                       keepdims=True))
        o_r[...] = jnp.concatenate([ls, l_r[...]], axis=1)
        h_r[...] = ls[:, :DOUT]

    return pl.pallas_call(
        body,
        grid=(N // BM,),
        in_specs=[pl.BlockSpec((1, BM, DOP), lambda i: (0, i, 0)),
                  pl.BlockSpec((1, BM, DOP), lambda i: (1, i, 0)),
                  pl.BlockSpec((BM, DOP), lambda i: (i, 0)),
                  pl.BlockSpec((BM, DOP), lambda i: (i, 0)),
                  pl.BlockSpec((1, DOP), lambda i: (0, 0)),
                  pl.BlockSpec((BM, 16), lambda i: (i, 0))],
        out_specs=[pl.BlockSpec((BM, DOP + 16), lambda i: (i, 0)),
                   pl.BlockSpec((BM, DOUT), lambda i: (i, 0))],
        out_shape=[jax.ShapeDtypeStruct((N, DOP + 16), _F32),
                   jax.ShapeDtypeStruct((N, DOUT), _F32)],
    )(acc2, acc2, y2, rr, b2p, labf)


def _select(t3, tnp):
    @functools.partial(
        pl.kernel,
        out_type=jax.ShapeDtypeStruct((NTP, DOP + 16), _F32),
        mesh=_vmesh(),
        compiler_params=_SC_PARAMS,
        scratch_types=[pltpu.VMEM((NTP // NW,), jnp.int32),
                       pltpu.VMEM((NTP // NW, DOP + 16), _F32),
                       pltpu.SemaphoreType.DMA],
    )
    def k(t3_hbm, tn_hbm, o_hbm, idx, buf, sem):
        c = lax.axis_index("c")
        s = lax.axis_index("s")
        w = c * NS + s
        bs = NTP // NW
        sl = pl.ds(w * bs, bs)
        pltpu.sync_copy(tn_hbm.at[sl], idx)
        pltpu.async_copy(t3_hbm.at[idx], buf, sem).wait()
        pltpu.sync_copy(buf, o_hbm.at[sl])

    return k(t3, tnp)


def _loss(sel):
    def body(s_ref, o_ref):
        s = s_ref[...]
        ls = s[:, :DOP]
        lbl = s[:, DOP:DOP + 1].astype(jnp.int32)
        col = lax.broadcasted_iota(jnp.int32, (NTP, DOP), 1)
        picked = jnp.sum(jnp.where(col == lbl, ls, 0.0), axis=1,
                         keepdims=True)
        rowv = lax.broadcasted_iota(jnp.int32, (NTP, 1), 0) < NT
        tot = jnp.sum(jnp.where(rowv, picked, 0.0), axis=0, keepdims=True)
        o_ref[...] = -tot / NT

    return pl.pallas_call(
        body,
        out_shape=jax.ShapeDtypeStruct((1, 1), _F32),
    )(sel)


def kernel(feats, edge_index, label, train_nodes, W1, b1, W2, b2):
    w2p = jnp.zeros((DIN, DOP), _F32).at[:, :DOUT].set(W2)
    b1r = b1[None, :]
    b2p = jnp.zeros((1, DOP), _F32).at[0, :DOUT].set(b2)
    labf = jnp.broadcast_to(label.astype(_F32)[:, None], (N, 16))
    tnp = jnp.concatenate(
        [train_nodes, jnp.zeros((NTP - NT,), jnp.int32)])

    accf, accd = _agg_deg(feats, edge_index)
    y2, rr = _layer1(accf, accd, feats, b1r, W1, w2p)
    acc2 = _agg(y2, edge_index)
    t3, h_out = _layer2_softmax(acc2, y2, rr, b2p, labf)
    sel = _select(t3, tnp)
    loss = _loss(sel)[0, 0]
    return (h_out, loss)
